# Initial kernel scaffold; baseline (speedup 1.0000x reference)
import jax
import jax.numpy as jnp
from jax.experimental import pallas as pl


def kernel(g, atom_types, edge_distances, emb, conv_params, Wa1, ba1, Wa2, ba2, Wr1, br1, Wr2, br2):
    raise NotImplementedError("write your pallas kernel here")


# speedup vs baseline: 3.3691x; 3.3691x over previous
import functools

import numpy as np
import jax
import jax.numpy as jnp
from jax import lax
from jax.experimental import pallas as pl
from jax.experimental.pallas import tpu as pltpu
from jax.experimental.pallas import tpu_sc as plsc

N = 200
E = N * (N - 1)
DIM = 64
NTYPE = 100
NRBF = 50
CUTOFF = 5.0
GAP = CUTOFF / (NRBF - 1)
COEF = -1.0 / GAP
LOG2 = float(np.log(2.0))

NWORK = 32
CHUNK = 128
NCHUNK = 10
PER_W = CHUNK * NCHUNK
EP = NWORK * PER_W
NACC = 208

EBLK = 2048
NEBLK = EP // EBLK

_F32 = jnp.float32


def _sp05(x):
    bx = 0.5 * x
    soft = jnp.maximum(bx, 0.0) + jnp.log1p(jnp.exp(-jnp.abs(bx)))
    return jnp.where(bx > 14.0, x, 2.0 * soft)


def _shift_sp(x):
    soft = jnp.maximum(x, 0.0) + jnp.log1p(jnp.exp(-jnp.abs(x)))
    return jnp.where(x > 20.0, x, soft) - LOG2


def _edge_dense_body(d_ref, wp1t_ref, bp1_ref, wp2t_ref, bp2_ref,
                     wr1rt_ref, br1_ref, e0_ref, e1_ref, e2_ref, r_ref):
    d = d_ref[...]
    cen = lax.broadcasted_iota(jnp.int32, (EBLK, NRBF), 1).astype(_F32) * GAP
    rbf = jnp.exp(COEF * (d - cen) ** 2)
    for l, eref in enumerate((e0_ref, e1_ref, e2_ref)):
        t = _sp05(jnp.dot(rbf, wp1t_ref[l], preferred_element_type=_F32)
                  + bp1_ref[l])
        eref[...] = jnp.dot(t, wp2t_ref[l], preferred_element_type=_F32) + bp2_ref[l]
    r_ref[...] = (jnp.dot(rbf, wr1rt_ref[...], preferred_element_type=_F32)
                  + br1_ref[...])


def _edge_dense(dp, wp1t, bp1, wp2t, bp2, wr1rt, br1):
    full = lambda s: pl.BlockSpec(s, lambda i: (0,) * len(s))
    outs = [jax.ShapeDtypeStruct((EP, DIM), _F32)] * 4
    return pl.pallas_call(
        _edge_dense_body,
        grid=(NEBLK,),
        in_specs=[
            pl.BlockSpec((EBLK, 1), lambda i: (i, 0)),
            full((3, NRBF, DIM)), full((3, 1, DIM)),
            full((3, DIM, DIM)), full((3, 1, DIM)),
            full((NRBF, DIM)), full((1, DIM)),
        ],
        out_specs=[pl.BlockSpec((EBLK, DIM), lambda i: (i, 0))] * 4,
        out_shape=outs,
    )(dp, wp1t, bp1, wp2t, bp2, wr1rt, br1)


def _node_init_body(at_ref, emb_ref, w1t_ref, h_ref, nw_ref):
    types = lax.broadcasted_iota(jnp.int32, (N, NTYPE), 1).astype(_F32)
    onehot = (at_ref[...] == types).astype(_F32)
    h = jnp.dot(onehot, emb_ref[...], preferred_element_type=_F32, precision=jax.lax.Precision.HIGHEST)
    h_ref[...] = h
    nw_ref[...] = jnp.dot(h, w1t_ref[...], preferred_element_type=_F32, precision=jax.lax.Precision.HIGHEST)


def _node_init(at_f, emb, w1t):
    return pl.pallas_call(
        _node_init_body,
        out_shape=[jax.ShapeDtypeStruct((N, DIM), _F32)] * 2,
    )(at_f, emb, w1t)


def _node_update_body(h_ref, ag_ref, wn1t_ref, bn1_ref, wn2t_ref, bn2_ref,
                      w1nt_ref, h_out, nw_out):
    agg = ag_ref[0] + ag_ref[1]
    t = _sp05(jnp.dot(agg, wn1t_ref[...], preferred_element_type=_F32, precision=jax.lax.Precision.HIGHEST)
              + bn1_ref[...])
    hn = h_ref[...] + jnp.dot(t, wn2t_ref[...], preferred_element_type=_F32, precision=jax.lax.Precision.HIGHEST) + bn2_ref[...]
    h_out[...] = hn
    nw_out[...] = jnp.dot(hn, w1nt_ref[...], preferred_element_type=_F32, precision=jax.lax.Precision.HIGHEST)


def _node_update(h, aggp, wn1t, bn1, wn2t, bn2, w1nt):
    return pl.pallas_call(
        _node_update_body,
        out_shape=[jax.ShapeDtypeStruct((N, DIM), _F32)] * 2,
    )(h, aggp, wn1t, bn1, wn2t, bn2, w1nt)


def _node_final_body(h_ref, ag_ref, wn1t_ref, bn1_ref, wn2t_ref, bn2_ref,
                     wa1t_ref, ba1_ref, wa2t_ref, ba2_ref, ha_out):
    agg = ag_ref[0] + ag_ref[1]
    t = _sp05(jnp.dot(agg, wn1t_ref[...], preferred_element_type=_F32, precision=jax.lax.Precision.HIGHEST)
              + bn1_ref[...])
    hn = h_ref[...] + jnp.dot(t, wn2t_ref[...], preferred_element_type=_F32, precision=jax.lax.Precision.HIGHEST) + bn2_ref[...]
    u = _shift_sp(jnp.dot(hn, wa1t_ref[...], preferred_element_type=_F32, precision=jax.lax.Precision.HIGHEST)
                  + ba1_ref[...])
    ha_out[...] = (jnp.dot(u, wa2t_ref[...], preferred_element_type=_F32, precision=jax.lax.Precision.HIGHEST)
                   + ba2_ref[...])


def _node_final(h, aggp, wn1t, bn1, wn2t, bn2, wa1t, ba1, wa2t, ba2):
    return pl.pallas_call(
        _node_final_body,
        out_shape=jax.ShapeDtypeStruct((N, 1), _F32),
    )(h, aggp, wn1t, bn1, wn2t, bn2, wa1t, ba1, wa2t, ba2)


def _readout_body(r_ref, iif_ref, jjf_ref, ha_ref, a_ref, b_ref,
                  wr2t_ref, br2_ref, out_ref):
    iota_n = lax.broadcasted_iota(jnp.int32, (EBLK, N), 1).astype(_F32)
    ohi = (iif_ref[...] == iota_n).astype(_F32)
    ohj = (jjf_ref[...] == iota_n).astype(_F32)
    ha = ha_ref[...]
    hai = jnp.dot(ohi, ha, preferred_element_type=_F32, precision=jax.lax.Precision.HIGHEST)
    haj = jnp.dot(ohj, ha, preferred_element_type=_F32, precision=jax.lax.Precision.HIGHEST)
    t = jnp.maximum(r_ref[...] + hai * a_ref[...] + haj * b_ref[...], 0.0)
    logits = jnp.dot(t, wr2t_ref[...], preferred_element_type=_F32, precision=jax.lax.Precision.HIGHEST) + br2_ref[...]
    m = jnp.max(logits, axis=1, keepdims=True)
    ex = jnp.exp(logits - m)
    out_ref[...] = ex / jnp.sum(ex, axis=1, keepdims=True)


def _readout(r, iif, jjf, ha, avec, bvec, wr2t, br2):
    full = lambda s: pl.BlockSpec(s, lambda i: (0,) * len(s))
    return pl.pallas_call(
        _readout_body,
        grid=(NEBLK,),
        in_specs=[
            pl.BlockSpec((EBLK, DIM), lambda i: (i, 0)),
            pl.BlockSpec((EBLK, 1), lambda i: (i, 0)),
            pl.BlockSpec((EBLK, 1), lambda i: (i, 0)),
            full((N, 1)), full((1, DIM)), full((1, DIM)),
            full((DIM, 2)), full((1, 2)),
        ],
        out_specs=pl.BlockSpec((EBLK, 2), lambda i: (i, 0)),
        out_shape=jax.ShapeDtypeStruct((EP, 2), _F32),
    )(r, iif, jjf, ha, avec, bvec, wr2t, br2)


def _sc_msgpass(nw, e, src_p, dst_p):
    mesh = plsc.VectorSubcoreMesh(core_axis_name="c", subcore_axis_name="s")

    @functools.partial(
        pl.kernel,
        out_type=jax.ShapeDtypeStruct((2, N, DIM), _F32),
        mesh=mesh,
        scratch_types=[
            pltpu.VMEM((CHUNK,), jnp.int32),
            pltpu.VMEM((CHUNK,), jnp.int32),
            pltpu.VMEM((CHUNK, DIM), _F32),
            pltpu.VMEM((CHUNK, DIM), _F32),
            pltpu.VMEM((NACC // 16, DIM), _F32),
            pltpu.VMEM_SHARED((NACC, DIM), _F32),
            pltpu.SemaphoreType.DMA,
        ],
        compiler_params=pltpu.CompilerParams(use_tc_tiling_on_sc=False),
    )
    def body(nw_hbm, e_hbm, src_hbm, dst_hbm, out_hbm,
             srcv, dstv, ev, mv, zv, aggsh, sem):
        cid = lax.axis_index("c")
        sid = lax.axis_index("s")
        wid = sid * 2 + cid
        zrows = NACC // 16
        zero = jnp.zeros((16,), _F32)
        for j in range(zrows):
            for c in range(DIM // 16):
                zv[j, pl.ds(c * 16, 16)] = zero
        pltpu.sync_copy(zv, aggsh.at[pl.ds(sid * zrows, zrows), :])
        plsc.subcore_barrier()
        base = wid * PER_W
        for k in range(NCHUNK):
            off = base + k * CHUNK
            pltpu.sync_copy(src_hbm.at[pl.ds(off, CHUNK)], srcv)
            pltpu.sync_copy(dst_hbm.at[pl.ds(off, CHUNK)], dstv)
            pltpu.async_copy(nw_hbm.at[srcv], mv, sem).wait()
            pltpu.sync_copy(e_hbm.at[pl.ds(off, CHUNK), :], ev)

            def mul_body(j, carry):
                for c in range(DIM // 16):
                    sl = pl.ds(c * 16, 16)
                    mv[j, sl] = mv[j, sl] * ev[j, sl]
                return carry

            lax.fori_loop(0, CHUNK, mul_body, 0)
            pltpu.sync_copy(mv, aggsh.at[dstv], add=True)
        plsc.subcore_barrier()

        @pl.when(sid == 0)
        def _():
            pltpu.sync_copy(aggsh.at[pl.ds(0, N), :], out_hbm.at[cid])

    return body(nw, e, src_p, dst_p)


def kernel(g, atom_types, edge_distances, emb, conv_params,
           Wa1, ba1, Wa2, ba2, Wr1, br1, Wr2, br2):
    src = g[0].astype(jnp.int32)
    dst = g[1].astype(jnp.int32)
    pad = EP - E
    src_p = jnp.concatenate([src, jnp.zeros((pad,), jnp.int32)])
    dst_p = jnp.concatenate([dst, jnp.full((pad,), N, jnp.int32)])
    dp = jnp.concatenate([edge_distances.astype(_F32),
                          jnp.zeros((pad, 1), _F32)], axis=0)

    wp1t = jnp.stack([p[1].T for p in conv_params])
    bp1 = jnp.stack([p[2].reshape(1, DIM) for p in conv_params])
    wp2t = jnp.stack([p[3].T for p in conv_params])
    bp2 = jnp.stack([p[4].reshape(1, DIM) for p in conv_params])
    wr1rt = Wr1[:, 2:].T
    br1r = br1.reshape(1, DIM)
    avec = Wr1[:, 0].reshape(1, DIM)
    bvec = Wr1[:, 1].reshape(1, DIM)

    e0, e1, e2, r_out = _edge_dense(dp, wp1t, bp1, wp2t, bp2, wr1rt, br1r)
    es = (e0, e1, e2)

    at_f = atom_types.astype(_F32).reshape(N, 1)
    h, nw = _node_init(at_f, emb, conv_params[0][0].T)

    for l in range(3):
        (W1, _, _, _, _, Wn1, bn1, Wn2, bn2) = conv_params[l]
        aggp = _sc_msgpass(nw, es[l], src_p, dst_p)
        if l < 2:
            w1nt = conv_params[l + 1][0].T
            h, nw = _node_update(h, aggp, Wn1.T, bn1.reshape(1, DIM),
                                 Wn2.T, bn2.reshape(1, DIM), w1nt)
        else:
            ha = _node_final(h, aggp, Wn1.T, bn1.reshape(1, DIM),
                             Wn2.T, bn2.reshape(1, DIM),
                             Wa1.T, ba1.reshape(1, DIM),
                             Wa2.T, ba2.reshape(1, 1))

    ii_np = np.repeat(np.arange(N), N)
    jj_np = np.tile(np.arange(N), N)
    mask = ii_np != jj_np
    iif_np = np.zeros((EP, 1), np.float32)
    jjf_np = np.zeros((EP, 1), np.float32)
    iif_np[:E, 0] = ii_np[mask]
    jjf_np[:E, 0] = jj_np[mask]

    out = _readout(r_out, jnp.asarray(iif_np), jnp.asarray(jjf_np), ha,
                   avec, bvec, Wr2.T, br2.reshape(1, 2))
    return out[:E]


# speedup vs baseline: 4.4029x; 1.3068x over previous
import functools

import numpy as np
import jax
import jax.numpy as jnp
from jax import lax
from jax.experimental import pallas as pl
from jax.experimental.pallas import tpu as pltpu
from jax.experimental.pallas import tpu_sc as plsc

N = 200
E = N * (N - 1)
DIM = 64
NTYPE = 100
NRBF = 50
CUTOFF = 5.0
GAP = CUTOFF / (NRBF - 1)
COEF = -1.0 / GAP
LOG2 = float(np.log(2.0))

NWORK = 32
CHUNK = 128
NCHUNK = 10
PER_W = CHUNK * NCHUNK
EP = NWORK * PER_W
NACC = 208

EBLK = 2048
NEBLK = EP // EBLK

_F32 = jnp.float32


def _sp05(x):
    bx = 0.5 * x
    soft = jnp.maximum(bx, 0.0) + jnp.log1p(jnp.exp(-jnp.abs(bx)))
    return jnp.where(bx > 14.0, x, 2.0 * soft)


def _shift_sp(x):
    soft = jnp.maximum(x, 0.0) + jnp.log1p(jnp.exp(-jnp.abs(x)))
    return jnp.where(x > 20.0, x, soft) - LOG2


def _edge_dense_body(d_ref, wp1t_ref, bp1_ref, wp2t_ref, bp2_ref,
                     wr1rt_ref, br1_ref, e0_ref, e1_ref, e2_ref, r_ref):
    d = d_ref[...]
    cen = lax.broadcasted_iota(jnp.int32, (EBLK, NRBF), 1).astype(_F32) * GAP
    rbf = jnp.exp(COEF * (d - cen) ** 2)
    for l, eref in enumerate((e0_ref, e1_ref, e2_ref)):
        t = _sp05(jnp.dot(rbf, wp1t_ref[l], preferred_element_type=_F32)
                  + bp1_ref[l])
        eref[...] = jnp.dot(t, wp2t_ref[l], preferred_element_type=_F32) + bp2_ref[l]
    r_ref[...] = (jnp.dot(rbf, wr1rt_ref[...], preferred_element_type=_F32)
                  + br1_ref[...])


def _edge_dense(dp, wp1t, bp1, wp2t, bp2, wr1rt, br1):
    full = lambda s: pl.BlockSpec(s, lambda i: (0,) * len(s))
    outs = [jax.ShapeDtypeStruct((EP, DIM), _F32)] * 4
    return pl.pallas_call(
        _edge_dense_body,
        grid=(NEBLK,),
        in_specs=[
            pl.BlockSpec((EBLK, 1), lambda i: (i, 0)),
            full((3, NRBF, DIM)), full((3, 1, DIM)),
            full((3, DIM, DIM)), full((3, 1, DIM)),
            full((NRBF, DIM)), full((1, DIM)),
        ],
        out_specs=[pl.BlockSpec((EBLK, DIM), lambda i: (i, 0))] * 4,
        out_shape=outs,
    )(dp, wp1t, bp1, wp2t, bp2, wr1rt, br1)


def _node_init_body(at_ref, emb_ref, w1t_ref, h_ref, nw_ref):
    types = lax.broadcasted_iota(jnp.int32, (N, NTYPE), 1).astype(_F32)
    onehot = (at_ref[...] == types).astype(_F32)
    h = jnp.dot(onehot, emb_ref[...], preferred_element_type=_F32, precision=jax.lax.Precision.HIGHEST)
    h_ref[...] = h
    nw_ref[...] = jnp.dot(h, w1t_ref[...], preferred_element_type=_F32, precision=jax.lax.Precision.HIGHEST)


def _node_init(at_f, emb, w1t):
    return pl.pallas_call(
        _node_init_body,
        out_shape=[jax.ShapeDtypeStruct((N, DIM), _F32)] * 2,
    )(at_f, emb, w1t)


def _node_update_body(h_ref, ag_ref, wn1t_ref, bn1_ref, wn2t_ref, bn2_ref,
                      w1nt_ref, h_out, nw_out):
    agg = ag_ref[0] + ag_ref[1]
    t = _sp05(jnp.dot(agg, wn1t_ref[...], preferred_element_type=_F32, precision=jax.lax.Precision.HIGHEST)
              + bn1_ref[...])
    hn = h_ref[...] + jnp.dot(t, wn2t_ref[...], preferred_element_type=_F32, precision=jax.lax.Precision.HIGHEST) + bn2_ref[...]
    h_out[...] = hn
    nw_out[...] = jnp.dot(hn, w1nt_ref[...], preferred_element_type=_F32, precision=jax.lax.Precision.HIGHEST)


def _node_update(h, aggp, wn1t, bn1, wn2t, bn2, w1nt):
    return pl.pallas_call(
        _node_update_body,
        out_shape=[jax.ShapeDtypeStruct((N, DIM), _F32)] * 2,
    )(h, aggp, wn1t, bn1, wn2t, bn2, w1nt)


def _node_final_body(h_ref, ag_ref, wn1t_ref, bn1_ref, wn2t_ref, bn2_ref,
                     wa1t_ref, ba1_ref, wa2t_ref, ba2_ref, ha_out):
    agg = ag_ref[0] + ag_ref[1]
    t = _sp05(jnp.dot(agg, wn1t_ref[...], preferred_element_type=_F32, precision=jax.lax.Precision.HIGHEST)
              + bn1_ref[...])
    hn = h_ref[...] + jnp.dot(t, wn2t_ref[...], preferred_element_type=_F32, precision=jax.lax.Precision.HIGHEST) + bn2_ref[...]
    u = _shift_sp(jnp.dot(hn, wa1t_ref[...], preferred_element_type=_F32, precision=jax.lax.Precision.HIGHEST)
                  + ba1_ref[...])
    ha_out[...] = (jnp.dot(u, wa2t_ref[...], preferred_element_type=_F32, precision=jax.lax.Precision.HIGHEST)
                   + ba2_ref[...])


def _node_final(h, aggp, wn1t, bn1, wn2t, bn2, wa1t, ba1, wa2t, ba2):
    return pl.pallas_call(
        _node_final_body,
        out_shape=jax.ShapeDtypeStruct((N, 1), _F32),
    )(h, aggp, wn1t, bn1, wn2t, bn2, wa1t, ba1, wa2t, ba2)


def _readout_body(r_ref, iif_ref, jjf_ref, ha_ref, a_ref, b_ref,
                  wr2t_ref, br2_ref, out_ref):
    iota_n = lax.broadcasted_iota(jnp.int32, (EBLK, N), 1).astype(_F32)
    ha = ha_ref[...]
    ohi = jnp.where(iif_ref[...] == iota_n, ha, 0.0)
    ohj = jnp.where(jjf_ref[...] == iota_n, ha, 0.0)
    hai = jnp.sum(ohi, axis=1, keepdims=True)
    haj = jnp.sum(ohj, axis=1, keepdims=True)
    t = jnp.maximum(r_ref[...] + hai * a_ref[...] + haj * b_ref[...], 0.0)
    logits = jnp.dot(t, wr2t_ref[...], preferred_element_type=_F32, precision=jax.lax.Precision.HIGHEST) + br2_ref[...]
    m = jnp.max(logits, axis=1, keepdims=True)
    ex = jnp.exp(logits - m)
    out_ref[...] = ex / jnp.sum(ex, axis=1, keepdims=True)


def _readout(r, iif, jjf, ha, avec, bvec, wr2t, br2):
    full = lambda s: pl.BlockSpec(s, lambda i: (0,) * len(s))
    return pl.pallas_call(
        _readout_body,
        grid=(NEBLK,),
        in_specs=[
            pl.BlockSpec((EBLK, DIM), lambda i: (i, 0)),
            pl.BlockSpec((EBLK, 1), lambda i: (i, 0)),
            pl.BlockSpec((EBLK, 1), lambda i: (i, 0)),
            full((1, N)), full((1, DIM)), full((1, DIM)),
            full((DIM, 2)), full((1, 2)),
        ],
        out_specs=pl.BlockSpec((EBLK, 2), lambda i: (i, 0)),
        out_shape=jax.ShapeDtypeStruct((EP, 2), _F32),
    )(r, iif, jjf, ha, avec, bvec, wr2t, br2)


def _sc_msgpass(nw, e, src2d, dst2d):
    mesh = plsc.VectorSubcoreMesh(core_axis_name="c", subcore_axis_name="s")

    @functools.partial(
        pl.kernel,
        out_type=jax.ShapeDtypeStruct((2, N, DIM), _F32),
        mesh=mesh,
        scratch_types=[
            pltpu.VMEM((NCHUNK, CHUNK), jnp.int32),
            pltpu.VMEM((NCHUNK, CHUNK), jnp.int32),
            pltpu.VMEM((2, CHUNK, DIM), _F32),
            pltpu.VMEM((2, CHUNK, DIM), _F32),
            pltpu.VMEM((NACC // 16, DIM), _F32),
            pltpu.VMEM_SHARED((NACC, DIM), _F32),
            pltpu.SemaphoreType.DMA, pltpu.SemaphoreType.DMA,
            pltpu.SemaphoreType.DMA, pltpu.SemaphoreType.DMA,
        ],
        compiler_params=pltpu.CompilerParams(use_tc_tiling_on_sc=False),
    )
    def body(nw_hbm, e_hbm, src_hbm, dst_hbm, out_hbm,
             srcv, dstv, ev, mv, zv, aggsh, esem0, esem1, gsem0, gsem1):
        cid = lax.axis_index("c")
        sid = lax.axis_index("s")
        wid = sid * 2 + cid
        esems = (esem0, esem1)
        gsems = (gsem0, gsem1)
        pltpu.sync_copy(src_hbm.at[pl.ds(wid * NCHUNK, NCHUNK), :], srcv)
        pltpu.sync_copy(dst_hbm.at[pl.ds(wid * NCHUNK, NCHUNK), :], dstv)
        zrows = NACC // 16
        zero = jnp.zeros((16,), _F32)
        for j in range(zrows):
            for c in range(DIM // 16):
                zv[j, pl.ds(c * 16, 16)] = zero
        pltpu.sync_copy(zv, aggsh.at[pl.ds(sid * zrows, zrows), :])
        plsc.subcore_barrier()
        base = wid * PER_W

        def issue(k, buf):
            off = base + k * CHUNK
            ec = pltpu.async_copy(e_hbm.at[pl.ds(off, CHUNK), :], ev.at[buf],
                                  esems[buf])
            gc = pltpu.async_copy(nw_hbm.at[srcv.at[k]], mv.at[buf], gsems[buf])
            return ec, gc

        pending = [None, None]
        pending[0] = issue(0, 0)
        for k in range(NCHUNK):
            buf = k & 1
            ec, gc = pending[buf]
            ec.wait()
            gc.wait()
            if k + 1 < NCHUNK:
                pending[(k + 1) & 1] = issue(k + 1, (k + 1) & 1)

            def mul_body(j, carry):
                for c in range(DIM // 16):
                    sl = pl.ds(c * 16, 16)
                    mv[buf, j, sl] = mv[buf, j, sl] * ev[buf, j, sl]
                return carry

            lax.fori_loop(0, CHUNK, mul_body, 0)
            pltpu.sync_copy(mv.at[buf], aggsh.at[dstv.at[k]], add=True)
        plsc.subcore_barrier()

        @pl.when(sid == 0)
        def _():
            pltpu.sync_copy(aggsh.at[pl.ds(0, N), :], out_hbm.at[cid])

    return body(nw, e, src2d, dst2d)


def kernel(g, atom_types, edge_distances, emb, conv_params,
           Wa1, ba1, Wa2, ba2, Wr1, br1, Wr2, br2):
    src = g[0].astype(jnp.int32)
    dst = g[1].astype(jnp.int32)
    pad = EP - E
    src2d = jnp.concatenate([src, jnp.zeros((pad,), jnp.int32)]
                            ).reshape(NWORK * NCHUNK, CHUNK)
    dst2d = jnp.concatenate([dst, jnp.full((pad,), N, jnp.int32)]
                            ).reshape(NWORK * NCHUNK, CHUNK)
    dp = jnp.concatenate([edge_distances.astype(_F32),
                          jnp.zeros((pad, 1), _F32)], axis=0)

    wp1t = jnp.stack([p[1].T for p in conv_params])
    bp1 = jnp.stack([p[2].reshape(1, DIM) for p in conv_params])
    wp2t = jnp.stack([p[3].T for p in conv_params])
    bp2 = jnp.stack([p[4].reshape(1, DIM) for p in conv_params])
    wr1rt = Wr1[:, 2:].T
    br1r = br1.reshape(1, DIM)
    avec = Wr1[:, 0].reshape(1, DIM)
    bvec = Wr1[:, 1].reshape(1, DIM)

    e0, e1, e2, r_out = _edge_dense(dp, wp1t, bp1, wp2t, bp2, wr1rt, br1r)
    es = (e0, e1, e2)

    at_f = atom_types.astype(_F32).reshape(N, 1)
    h, nw = _node_init(at_f, emb, conv_params[0][0].T)

    for l in range(3):
        (W1, _, _, _, _, Wn1, bn1, Wn2, bn2) = conv_params[l]
        aggp = _sc_msgpass(nw, es[l], src2d, dst2d)
        if l < 2:
            w1nt = conv_params[l + 1][0].T
            h, nw = _node_update(h, aggp, Wn1.T, bn1.reshape(1, DIM),
                                 Wn2.T, bn2.reshape(1, DIM), w1nt)
        else:
            ha = _node_final(h, aggp, Wn1.T, bn1.reshape(1, DIM),
                             Wn2.T, bn2.reshape(1, DIM),
                             Wa1.T, ba1.reshape(1, DIM),
                             Wa2.T, ba2.reshape(1, 1))

    ii_np = np.repeat(np.arange(N), N)
    jj_np = np.tile(np.arange(N), N)
    mask = ii_np != jj_np
    iif_np = np.zeros((EP, 1), np.float32)
    jjf_np = np.zeros((EP, 1), np.float32)
    iif_np[:E, 0] = ii_np[mask]
    jjf_np[:E, 0] = jj_np[mask]

    out = _readout(r_out, jnp.asarray(iif_np), jnp.asarray(jjf_np), ha.T,
                   avec, bvec, Wr2.T, br2.reshape(1, 2))
    return out[:E]
